# baseline (speedup 1.0000x reference)
import functools

import jax
import jax.numpy as jnp
from jax import lax
from jax.experimental import pallas as pl
from jax.experimental.pallas import tpu as pltpu
from jax.experimental.pallas import tpu_sc as plsc

NO_OF_LEVELS = 16
HASHMAP_SIZE = 2 ** 19
N_POINTS = 524288
NW = 32
PTS_PER_W = N_POINTS // NW
C = 256
GROUPS = C // 16
MASK = HASHMAP_SIZE - 1
P1 = -1640531535
P2 = 805459861

_CORNERS = [(dx, dy, dz) for dx in (0, 1) for dy in (0, 1) for dz in (0, 1)]


def _body(x0, x1, x2, res, table, out,
          spt, xb0, xb1, xb2,
          wxb0, wyb0, wzb0, wxb1, wyb1, wzb1,
          idxb0, idxb1, subb0, subb1, rowsb0, rowsb1,
          outb, resb, semA, semB):
    wid = lax.axis_index("s") * 2 + lax.axis_index("c")
    base0 = wid * PTS_PER_W
    iota = lax.broadcasted_iota(jnp.int32, (16,), 0)
    one16 = jnp.ones((16,), jnp.int32)

    pltpu.sync_copy(res, resb)

    sid = lax.axis_index("s")
    slice_rows = (HASHMAP_SIZE // 4) // 16
    pltpu.sync_copy(table.at[pl.ds(sid * slice_rows, slice_rows)],
                    spt.at[pl.ds(sid * slice_rows, slice_rows)])
    plsc.subcore_barrier()

    bufs = (
        (idxb0, subb0, rowsb0, wxb0, wyb0, wzb0, semA),
        (idxb1, subb1, rowsb1, wxb1, wyb1, wzb1, semB),
    )

    def hash_level(lvl, bi):
        idxb, subb, _, wxb, wyb, wzb, _ = bufs[bi]
        res_s = resb[pl.ds(lvl, 16)][0]

        def hash_group(g, _):
            p0 = g * 16
            sx = xb0[pl.ds(p0, 16)] * res_s
            sy = xb1[pl.ds(p0, 16)] * res_s
            sz = xb2[pl.ds(p0, 16)] * res_s
            ix = sx.astype(jnp.int32)
            iy = sy.astype(jnp.int32)
            iz = sz.astype(jnp.int32)
            wxb[pl.ds(p0, 16)] = sx - ix.astype(jnp.float32)
            wyb[pl.ds(p0, 16)] = sy - iy.astype(jnp.float32)
            wzb[pl.ds(p0, 16)] = sz - iz.astype(jnp.float32)
            hy0 = iy * P1
            hy1 = hy0 + P1
            hz0 = iz * P2
            hz1 = hz0 + P2
            hx = (ix, ix + 1)
            hy = (hy0, hy1)
            hz = (hz0, hz1)
            for j, (dx, dy, dz) in enumerate(_CORNERS):
                h = (hx[dx] ^ hy[dy] ^ hz[dz]) & MASK
                lin = j * C + p0
                idxb[pl.ds(lin, 16)] = h >> 2
                subb[pl.ds(lin, 16)] = (h & 3) * 2
            return 0

        lax.fori_loop(0, GROUPS, hash_group, 0)

    def fire_level(bi):
        idxb, _, rowsb, _, _, _, sem = bufs[bi]
        return pltpu.async_copy(spt.at[idxb], rowsb, sem)

    def interp_level(lvl, bi, cp):
        _, subb, rowsb, wxb, wyb, wzb, _ = bufs[bi]
        cp.wait()
        col0 = jnp.full((16,), 2 * lvl, jnp.int32)
        col1 = col0 + 1

        def interp_group(g, _):
            p0 = g * 16
            wx = wxb[pl.ds(p0, 16)]
            wy = wyb[pl.ds(p0, 16)]
            wz = wzb[pl.ds(p0, 16)]
            u0 = 1.0 - wx
            v0 = 1.0 - wy
            q0 = 1.0 - wz
            t00 = v0 * q0
            t01 = v0 * wz
            t10 = wy * q0
            t11 = wy * wz
            cw = (u0 * t00, u0 * t01, u0 * t10, u0 * t11,
                  wx * t00, wx * t01, wx * t10, wx * t11)
            pvec = p0 + iota
            acc0 = jnp.zeros((16,), jnp.float32)
            acc1 = jnp.zeros((16,), jnp.float32)
            for j in range(8):
                rowv = pvec + (j * C)
                sub = subb[pl.ds(j * C + p0, 16)]
                f0 = plsc.load_gather(rowsb, [rowv, sub])
                f1 = plsc.load_gather(rowsb, [rowv, sub + one16])
                acc0 = acc0 + cw[j] * f0
                acc1 = acc1 + cw[j] * f1
            plsc.store_scatter(outb, [pvec, col0], acc0)
            plsc.store_scatter(outb, [pvec, col1], acc1)
            return 0

        lax.fori_loop(0, GROUPS, interp_group, 0)

    def chunk_body(ci, _):
        base = base0 + ci * C
        pltpu.sync_copy(x0.at[pl.ds(base, C)], xb0)
        pltpu.sync_copy(x1.at[pl.ds(base, C)], xb1)
        pltpu.sync_copy(x2.at[pl.ds(base, C)], xb2)

        hash_level(0, 0)
        cp = fire_level(0)
        for lvl in range(1, NO_OF_LEVELS):
            bi = lvl & 1
            hash_level(lvl, bi)
            cp_next = fire_level(bi)
            interp_level(lvl - 1, 1 - bi, cp)
            cp = cp_next
        interp_level(NO_OF_LEVELS - 1, 1, cp)

        pltpu.sync_copy(outb, out.at[pl.ds(base, C)])
        return 0

    lax.fori_loop(0, PTS_PER_W // C, chunk_body, 0)


@jax.jit
def _run(x0, x1, x2, res, table):
    mesh = plsc.VectorSubcoreMesh(core_axis_name="c", subcore_axis_name="s")
    f = functools.partial(
        pl.kernel,
        mesh=mesh,
        compiler_params=pltpu.CompilerParams(
            needs_layout_passes=False, use_tc_tiling_on_sc=False),
        out_type=jax.ShapeDtypeStruct((N_POINTS, 2 * NO_OF_LEVELS),
                                      jnp.float32),
        scratch_types=[
            pltpu.VMEM_SHARED((HASHMAP_SIZE // 4, 8), jnp.float32),
            pltpu.VMEM((C,), jnp.float32),
            pltpu.VMEM((C,), jnp.float32),
            pltpu.VMEM((C,), jnp.float32),
            pltpu.VMEM((C,), jnp.float32),
            pltpu.VMEM((C,), jnp.float32),
            pltpu.VMEM((C,), jnp.float32),
            pltpu.VMEM((C,), jnp.float32),
            pltpu.VMEM((C,), jnp.float32),
            pltpu.VMEM((C,), jnp.float32),
            pltpu.VMEM((8 * C,), jnp.int32),
            pltpu.VMEM((8 * C,), jnp.int32),
            pltpu.VMEM((8 * C,), jnp.int32),
            pltpu.VMEM((8 * C,), jnp.int32),
            pltpu.VMEM((8 * C, 8), jnp.float32),
            pltpu.VMEM((8 * C, 8), jnp.float32),
            pltpu.VMEM((C, 2 * NO_OF_LEVELS), jnp.float32),
            pltpu.VMEM((2 * NO_OF_LEVELS,), jnp.float32),
            pltpu.SemaphoreType.DMA,
            pltpu.SemaphoreType.DMA,
        ],
    )(_body)
    return f(x0, x1, x2, res, table)


def kernel(x, embeddings):
    b = jnp.exp((jnp.log(jnp.float32(512.0)) - jnp.log(jnp.float32(16.0)))
                / jnp.float32(NO_OF_LEVELS - 1))
    res = jnp.stack([jnp.floor(jnp.float32(16.0) * (b ** i))
                     for i in range(NO_OF_LEVELS)]
                    + [jnp.float32(0.0)] * NO_OF_LEVELS)
    return _run(x[:, 0], x[:, 1], x[:, 2], res,
                embeddings.reshape(HASHMAP_SIZE // 4, 8))
